# Initial kernel scaffold; baseline (speedup 1.0000x reference)
import jax
import jax.numpy as jnp
from jax.experimental import pallas as pl


def kernel(x, spatial_idx, spatial_wgt, alignment, dist, Wq, Wk, Wv, Wp, bp):
    raise NotImplementedError("write your pallas kernel here")


# speedup vs baseline: 13.3432x; 13.3432x over previous
import functools
import math

import jax
import jax.numpy as jnp
from jax import lax
from jax.experimental import pallas as pl


def _attn_kernel(x_ref, idx_ref, wgt_ref, ali_ref, dst_ref,
                 wq_ref, wk0_ref, wv0_ref, wx_ref, wp_ref, bp_ref,
                 out_ref, *, S, C, H, K):
    d = C // H
    f32 = jnp.float32
    xb = x_ref[0]
    wq = wq_ref[...]
    xk = jnp.dot(xb, wk0_ref[...], preferred_element_type=f32)
    xv = jnp.dot(xb, wv0_ref[...], preferred_element_type=f32)
    q = jnp.dot(xb, wq, preferred_element_type=f32)
    xkv = jnp.concatenate([xk, xv], axis=1).astype(jnp.bfloat16)

    idxb = idx_ref[0]
    alib = ali_ref[0]
    dstb = dst_ref[0]
    lw = jnp.log(wgt_ref[0] + 1e-6)

    iota_j = lax.broadcasted_iota(jnp.int32, (S, S), 1)
    e_blocks = [(idxb[:, k:k + 1] == iota_j).astype(jnp.bfloat16) for k in range(K)]
    E = jnp.concatenate(e_blocks, axis=0)
    G = jnp.dot(E, xkv, preferred_element_type=f32)

    hb = lax.broadcasted_iota(jnp.int32, (C, H), 0) // d
    hcol = lax.broadcasted_iota(jnp.int32, (C, H), 1)
    bd = (hb == hcol).astype(f32)
    bdT = bd.T

    c1 = jnp.dot(q * wx_ref[0:1, :], bd, preferred_element_type=f32)
    c2 = jnp.dot(q * wx_ref[1:2, :], bd, preferred_element_type=f32)

    scale = 1.0 / math.sqrt(d)
    den = jnp.zeros((S, H), f32)
    num = jnp.zeros((S, C), f32)
    for k in range(K):
        Gk = G[k * S:(k + 1) * S]
        Kk = Gk[:, :C]
        Vk = (Gk[:, C:]
              + alib[:, k:k + 1] * wx_ref[2:3, :]
              + dstb[:, k:k + 1] * wx_ref[3:4, :])
        logit = jnp.dot(Kk * q, bd, preferred_element_type=f32)
        logit = (logit + alib[:, k:k + 1] * c1 + dstb[:, k:k + 1] * c2) * scale
        logit = logit + lw[:, k:k + 1]
        p = jnp.exp(logit)
        den = den + p
        p_exp = jnp.dot(p, bdT, preferred_element_type=f32)
        num = num + p_exp * Vk
    den_exp = jnp.dot(den, bdT, preferred_element_type=f32)
    out = num / den_exp
    out = jnp.dot(out, wp_ref[...], preferred_element_type=f32) + bp_ref[0:1, :]
    out_ref[0] = out


def kernel(x, spatial_idx, spatial_wgt, alignment, dist, Wq, Wk, Wv, Wp, bp):
    B, S, T, C = x.shape
    K = spatial_idx.shape[-1]
    H = 4
    BT = B * T
    f32 = jnp.float32

    x_ = jnp.transpose(x, (0, 2, 1, 3)).reshape(BT, S, C)
    idx = spatial_idx.reshape(BT, S, K).astype(jnp.int32)
    wgt = spatial_wgt.reshape(BT, S, K)
    ali = alignment.reshape(BT, S, K)
    dst = dist.reshape(BT, S, K)

    wx = jnp.concatenate([Wk[C:C + 2], Wv[C:C + 2],
                          jnp.zeros((4, C), f32)], axis=0)
    bp_pad = jnp.concatenate([bp.reshape(1, C), jnp.zeros((7, C), f32)], axis=0)

    grid = (BT,)
    bspec_bt = lambda: pl.BlockSpec((1, S, C), lambda i: (i, 0, 0))
    bspec_sk = lambda: pl.BlockSpec((1, S, K), lambda i: (i, 0, 0))
    bspec_w = lambda shape: pl.BlockSpec(shape, lambda i: (0, 0))

    out = pl.pallas_call(
        functools.partial(_attn_kernel, S=S, C=C, H=H, K=K),
        grid=grid,
        in_specs=[
            bspec_bt(),
            bspec_sk(),
            bspec_sk(),
            bspec_sk(),
            bspec_sk(),
            bspec_w((C, C)),
            bspec_w((C, C)),
            bspec_w((C, C)),
            bspec_w((8, C)),
            bspec_w((C, C)),
            bspec_w((8, C)),
        ],
        out_specs=bspec_bt(),
        out_shape=jax.ShapeDtypeStruct((BT, S, C), f32),
    )(x_, idx, wgt, ali, dst, Wq, Wk[:C], Wv[:C], wx, Wp, bp_pad)

    return out.reshape(B, T, S, C).transpose(0, 2, 1, 3)
